# baseline (speedup 1.0000x reference)
import jax
import jax.numpy as jnp
from jax import lax
from jax.experimental import pallas as pl
from jax.experimental.pallas import tpu as pltpu
from jax.experimental.pallas import tpu_sc as plsc

_N_FIELDS = 26
_VOCAB = 50
_EMBED_DIM = 32


def _body(sample_hbm, table_hbm, out_hbm, idx_s, sem):
    pltpu.sync_copy(sample_hbm, idx_s)
    copies = []
    for i in range(_N_FIELDS):
        copies.append(
            pltpu.make_async_copy(
                table_hbm.at[i, idx_s[i]], out_hbm.at[i], sem
            )
        )
        copies[-1].start()
    for c in copies:
        c.wait()


_gather = pl.kernel(
    _body,
    out_type=jax.ShapeDtypeStruct((_N_FIELDS, _EMBED_DIM), jnp.float32),
    mesh=plsc.ScalarSubcoreMesh(axis_name="c", num_cores=1),
    scratch_types=[
        pltpu.SMEM((_N_FIELDS,), jnp.int32),
        pltpu.SemaphoreType.DMA,
    ],
    compiler_params=pltpu.CompilerParams(use_tc_tiling_on_sc=False),
)


@jax.jit
def kernel(sample, tables):
    return _gather(sample, tables).reshape(-1)


# speedup vs baseline: 1.0102x; 1.0102x over previous
import jax
import jax.numpy as jnp
from jax import lax
from jax.experimental import pallas as pl
from jax.experimental.pallas import tpu as pltpu
from jax.experimental.pallas import tpu_sc as plsc

_N_FIELDS = 26
_VOCAB = 50
_EMBED_DIM = 32


def _body(sample_hbm, table_hbm, out_hbm, idx_s, sem):
    pltpu.sync_copy(sample_hbm, idx_s)
    copies = []
    for i in range(_N_FIELDS):
        copies.append(
            pltpu.make_async_copy(
                table_hbm.at[i, idx_s[i]],
                out_hbm.at[pl.ds(i * _EMBED_DIM, _EMBED_DIM)],
                sem,
            )
        )
        copies[-1].start()
    for c in copies:
        c.wait()


_gather = pl.kernel(
    _body,
    out_type=jax.ShapeDtypeStruct((_N_FIELDS * _EMBED_DIM,), jnp.float32),
    mesh=plsc.ScalarSubcoreMesh(axis_name="c", num_cores=1),
    scratch_types=[
        pltpu.SMEM((_N_FIELDS,), jnp.int32),
        pltpu.SemaphoreType.DMA,
    ],
    compiler_params=pltpu.CompilerParams(use_tc_tiling_on_sc=False),
)


@jax.jit
def kernel(sample, tables):
    return _gather(sample, tables)


# speedup vs baseline: 1.0180x; 1.0077x over previous
import jax
import jax.numpy as jnp
from jax import lax
from jax.experimental import pallas as pl
from jax.experimental.pallas import tpu as pltpu
from jax.experimental.pallas import tpu_sc as plsc

_N_FIELDS = 26
_VOCAB = 50
_EMBED_DIM = 32


def _body(sample_hbm, table_hbm, out_hbm, idx_s, sem):
    pltpu.sync_copy(sample_hbm, idx_s)

    def issue(i, carry):
        pltpu.make_async_copy(
            table_hbm.at[i, idx_s[i]], out_hbm.at[i], sem
        ).start()
        return carry

    lax.fori_loop(0, _N_FIELDS, issue, 0)
    pltpu.make_async_copy(table_hbm.at[:, 0], out_hbm, sem).wait()


_gather = pl.kernel(
    _body,
    out_type=jax.ShapeDtypeStruct((_N_FIELDS, _EMBED_DIM), jnp.float32),
    mesh=plsc.ScalarSubcoreMesh(axis_name="c", num_cores=1),
    scratch_types=[
        pltpu.SMEM((_N_FIELDS,), jnp.int32),
        pltpu.SemaphoreType.DMA,
    ],
    compiler_params=pltpu.CompilerParams(use_tc_tiling_on_sc=False),
)


@jax.jit
def kernel(sample, tables):
    return _gather(sample, tables).reshape(-1)
